# baseline (speedup 1.0000x reference)
import functools

import jax
import jax.numpy as jnp
from jax import lax
from jax.experimental import pallas as pl
from jax.experimental.pallas import tpu as pltpu
from jax.experimental.pallas import tpu_sc as plsc

_B = 32
_A = 102
_H = 224
_S = _H * _H
_R = 48
_NCH = _B * _A // _R
_NBUF = 4


def _pool_body(x_hbm, o_ref, y_hbm, *scratch):
    bufs = scratch[:_NBUF]
    isems = scratch[_NBUF:2 * _NBUF]
    osems = scratch[2 * _NBUF:]

    def cp_in(j, slot):
        return pltpu.make_async_copy(
            x_hbm.at[pl.ds(j * _R, _R)], bufs[slot], isems[slot])

    def cp_out(j, slot):
        return pltpu.make_async_copy(
            bufs[slot], y_hbm.at[pl.ds(j * _R, _R)], osems[slot])

    for s in range(_NBUF):
        cp_in(s, s).start()

    def outer(o, carry):
        base = o * _NBUF
        for b in range(_NBUF):
            i = base + b
            cp_in(i, b).wait()
            cp_out(i, b).start()
            partial = jnp.sum(bufs[b][...], axis=1)
            s = jnp.sum(partial, axis=1, keepdims=True) * (1.0 / _S)
            o_ref[pl.ds(i * _R, _R), :] = s
            nxt = i + _NBUF

            @pl.when(nxt < _NCH)
            def _():
                cp_out(i, b).wait()
                cp_in(nxt, b).start()
        return carry

    lax.fori_loop(0, _NCH // _NBUF, outer, 0)

    for s in range(_NBUF):
        cp_out(_NCH - _NBUF + s, s).wait()


_AP = 112
_NCHK = _AP // 16


def _sc_gate_body(x_hbm, w_hbm, out_hbm, xv, wv, ov):
    wid = lax.axis_index("s") * 2 + lax.axis_index("c")
    pltpu.sync_copy(x_hbm.at[wid], xv)
    pltpu.sync_copy(w_hbm, wv)
    lane = lax.iota(jnp.int32, 16)
    keys = []
    votes = []
    dense = jnp.zeros((16,), jnp.float32)
    for c in range(_NCHK):
        xc = xv[pl.ds(c * 16, 16)]
        wc = wv[pl.ds(c * 16, 16)]
        vote = xc * wc
        valid = (lane + c * 16) < _A
        keys.append(jnp.where(valid, jnp.abs(vote), -1.0))
        votes.append(vote)
        dense = dense + jnp.where(valid, vote, 0.0)
    bk, bv = plsc.sort_key_val(keys[0], votes[0], descending=True)
    low = lane < 8
    for c in range(1, _NCHK):
        ck, cv = plsc.sort_key_val(keys[c], votes[c], descending=True)
        mk = jnp.where(low, bk, jnp.flip(ck, axis=0))
        mv = jnp.where(low, bv, jnp.flip(cv, axis=0))
        bk, bv = plsc.sort_key_val(mk, mv, descending=True)
    pfx = plsc.cumsum(bv)
    dsum = jnp.sum(dense)
    out = jnp.where(lane == 8, jnp.full((16,), 1.0) * dsum, pfx) + 0.5
    ov[...] = out
    pltpu.sync_copy(ov, out_hbm.at[wid])


def kernel(maps, W2):
    n = _B * _A
    maps3 = maps.reshape(n, _H, _H)
    sums, maps_out = pl.pallas_call(
        _pool_body,
        in_specs=[pl.BlockSpec(memory_space=pl.ANY)],
        out_specs=[
            pl.BlockSpec(memory_space=pltpu.MemorySpace.VMEM),
            pl.BlockSpec(memory_space=pl.ANY),
        ],
        out_shape=[
            jax.ShapeDtypeStruct((n, 1), jnp.float32),
            jax.ShapeDtypeStruct((n, _H, _H), jnp.float32),
        ],
        scratch_shapes=(
            [pltpu.VMEM((_R, _H, _H), jnp.float32)] * _NBUF
            + [pltpu.SemaphoreType.DMA] * (2 * _NBUF)
        ),
    )(maps3)
    x_sun = sums.reshape(_B, _A)

    x_pad = jnp.pad(x_sun, ((0, 0), (0, _AP - _A)))
    w_pad = jnp.pad(W2.reshape(_A), (0, _AP - _A))
    mesh = plsc.VectorSubcoreMesh(core_axis_name="c", subcore_axis_name="s")
    gate = functools.partial(
        pl.kernel,
        mesh=mesh,
        out_type=jax.ShapeDtypeStruct((_B, 16), jnp.float32),
        scratch_types=[
            pltpu.VMEM((_AP,), jnp.float32),
            pltpu.VMEM((_AP,), jnp.float32),
            pltpu.VMEM((16,), jnp.float32),
        ],
        compiler_params=pltpu.CompilerParams(needs_layout_passes=False),
    )(_sc_gate_body)
    son = gate(x_pad, w_pad)
    x_son = son[:, :9].T.reshape(9, _B, 1)
    return (x_sun, x_son, maps_out.reshape(_B, _A, _H, _H))


# speedup vs baseline: 1.0003x; 1.0003x over previous
import functools

import jax
import jax.numpy as jnp
from jax import lax
from jax.experimental import pallas as pl
from jax.experimental.pallas import tpu as pltpu
from jax.experimental.pallas import tpu_sc as plsc

_B = 32
_A = 102
_H = 224
_S = _H * _H
_R = 48
_NCH = _B * _A // _R
_NBUF = 4


def _pool_body(x_hbm, o_ref, y_hbm, *scratch):
    bufs = scratch[:_NBUF]
    isems = scratch[_NBUF:2 * _NBUF]
    osems = scratch[2 * _NBUF:]

    def cp_in(j, slot):
        return pltpu.make_async_copy(
            x_hbm.at[pl.ds(j * _R, _R)], bufs[slot], isems[slot])

    def cp_out(j, slot):
        return pltpu.make_async_copy(
            bufs[slot], y_hbm.at[pl.ds(j * _R, _R)], osems[slot])

    for s in range(_NBUF):
        cp_in(s, s).start()

    def outer(o, carry):
        base = o * _NBUF
        for b in range(_NBUF):
            i = base + b
            cp_in(i, b).wait()
            cp_out(i, b).start()
            partial = jnp.sum(bufs[b][...], axis=1)
            s = jnp.sum(partial, axis=1, keepdims=True) * (1.0 / _S)
            o_ref[pl.ds(i * _R, _R), :] = s
            nxt = i + _NBUF

            @pl.when(nxt < _NCH)
            def _():
                cp_out(i, b).wait()
                cp_in(nxt, b).start()
        return carry

    lax.fori_loop(0, _NCH // _NBUF, outer, 0)

    for s in range(_NBUF):
        cp_out(_NCH - _NBUF + s, s).wait()


_AP = 112
_NCHK = _AP // 16


def _sc_gate_body(x_hbm, w_hbm, out_hbm, xv, wv, ov):
    wid = lax.axis_index("s") * 2 + lax.axis_index("c")
    pltpu.sync_copy(x_hbm, xv.at[pl.ds(0, _B * _A)])
    pltpu.sync_copy(w_hbm, wv.at[pl.ds(0, _A)])
    lane = lax.iota(jnp.int32, 16)
    base = wid * _A
    keys = []
    votes = []
    dense = jnp.zeros((16,), jnp.float32)
    for c in range(_NCHK):
        xc = xv[pl.ds(base + c * 16, 16)]
        wc = wv[pl.ds(c * 16, 16)]
        vote = xc * wc
        valid = (lane + c * 16) < _A
        keys.append(jnp.where(valid, jnp.abs(vote), -1.0))
        votes.append(jnp.where(valid, vote, 0.0))
        dense = dense + jnp.where(valid, vote, 0.0)
    bk, bv = plsc.sort_key_val(keys[0], votes[0], descending=True)
    low = lane < 8
    for c in range(1, _NCHK):
        ck, cv = plsc.sort_key_val(keys[c], votes[c], descending=True)
        mk = jnp.where(low, bk, jnp.flip(ck, axis=0))
        mv = jnp.where(low, bv, jnp.flip(cv, axis=0))
        bk, bv = plsc.sort_key_val(mk, mv, descending=True)
    pfx = plsc.cumsum(bv)
    dsum = jnp.sum(dense)
    out = jnp.where(lane == 8, jnp.full((16,), 1.0) * dsum, pfx) + 0.5
    ov[...] = out
    pltpu.sync_copy(ov, out_hbm.at[wid])


def kernel(maps, W2):
    n = _B * _A
    maps3 = maps.reshape(n, _H, _H)
    sums, maps_out = pl.pallas_call(
        _pool_body,
        in_specs=[pl.BlockSpec(memory_space=pl.ANY)],
        out_specs=[
            pl.BlockSpec(memory_space=pltpu.MemorySpace.VMEM),
            pl.BlockSpec(memory_space=pl.ANY),
        ],
        out_shape=[
            jax.ShapeDtypeStruct((n, 1), jnp.float32),
            jax.ShapeDtypeStruct((n, _H, _H), jnp.float32),
        ],
        scratch_shapes=(
            [pltpu.VMEM((_R, _H, _H), jnp.float32)] * _NBUF
            + [pltpu.SemaphoreType.DMA] * (2 * _NBUF)
        ),
    )(maps3)
    x_sun = sums.reshape(_B, _A)

    mesh = plsc.VectorSubcoreMesh(core_axis_name="c", subcore_axis_name="s")
    gate = functools.partial(
        pl.kernel,
        mesh=mesh,
        out_type=jax.ShapeDtypeStruct((_B, 16), jnp.float32),
        scratch_types=[
            pltpu.VMEM((_B * _A + _AP,), jnp.float32),
            pltpu.VMEM((_AP,), jnp.float32),
            pltpu.VMEM((16,), jnp.float32),
        ],
        compiler_params=pltpu.CompilerParams(needs_layout_passes=False),
    )(_sc_gate_body)
    son = gate(sums.reshape(n), W2.reshape(_A))
    x_son = son[:, :9].T.reshape(9, _B, 1)
    return (x_sun, x_son, maps_out.reshape(_B, _A, _H, _H))


# speedup vs baseline: 1.0020x; 1.0017x over previous
import functools

import jax
import jax.numpy as jnp
from jax import lax
from jax.experimental import pallas as pl
from jax.experimental.pallas import tpu as pltpu
from jax.experimental.pallas import tpu_sc as plsc

_B = 32
_A = 102
_H = 224
_S = _H * _H
_R = 96
_NCH = _B * _A // _R
_NBUF = 2


def _pool_body(x_hbm, o_ref, y_hbm, *scratch):
    bufs = scratch[:_NBUF]
    isems = scratch[_NBUF:2 * _NBUF]
    osems = scratch[2 * _NBUF:]

    def cp_in(j, slot):
        return pltpu.make_async_copy(
            x_hbm.at[pl.ds(j * _R, _R)], bufs[slot], isems[slot])

    def cp_out(j, slot):
        return pltpu.make_async_copy(
            bufs[slot], y_hbm.at[pl.ds(j * _R, _R)], osems[slot])

    for s in range(_NBUF):
        cp_in(s, s).start()

    def outer(o, carry):
        base = o * _NBUF
        for b in range(_NBUF):
            i = base + b
            cp_in(i, b).wait()
            cp_out(i, b).start()
            partial = jnp.sum(bufs[b][...], axis=1)
            s = jnp.sum(partial, axis=1, keepdims=True) * (1.0 / _S)
            o_ref[pl.ds(i * _R, _R), :] = s
            nxt = i + _NBUF

            @pl.when(nxt < _NCH)
            def _():
                cp_out(i, b).wait()
                cp_in(nxt, b).start()
        return carry

    lax.fori_loop(0, _NCH // _NBUF, outer, 0)

    for s in range(_NBUF):
        cp_out(_NCH - _NBUF + s, s).wait()


_AP = 112
_NCHK = _AP // 16


def _sc_gate_body(x_hbm, w_hbm, out_hbm, xv, wv, ov):
    wid = lax.axis_index("s") * 2 + lax.axis_index("c")
    pltpu.sync_copy(x_hbm, xv.at[pl.ds(0, _B * _A)])
    pltpu.sync_copy(w_hbm, wv.at[pl.ds(0, _A)])
    lane = lax.iota(jnp.int32, 16)
    base = wid * _A
    keys = []
    votes = []
    dense = jnp.zeros((16,), jnp.float32)
    for c in range(_NCHK):
        xc = xv[pl.ds(base + c * 16, 16)]
        wc = wv[pl.ds(c * 16, 16)]
        vote = xc * wc
        valid = (lane + c * 16) < _A
        keys.append(jnp.where(valid, jnp.abs(vote), -1.0))
        votes.append(jnp.where(valid, vote, 0.0))
        dense = dense + jnp.where(valid, vote, 0.0)
    bk, bv = plsc.sort_key_val(keys[0], votes[0], descending=True)
    low = lane < 8
    for c in range(1, _NCHK):
        ck, cv = plsc.sort_key_val(keys[c], votes[c], descending=True)
        mk = jnp.where(low, bk, jnp.flip(ck, axis=0))
        mv = jnp.where(low, bv, jnp.flip(cv, axis=0))
        bk, bv = plsc.sort_key_val(mk, mv, descending=True)
    pfx = plsc.cumsum(bv)
    dsum = jnp.sum(dense)
    out = jnp.where(lane == 8, jnp.full((16,), 1.0) * dsum, pfx) + 0.5
    ov[...] = out
    pltpu.sync_copy(ov, out_hbm.at[wid])


def kernel(maps, W2):
    n = _B * _A
    maps3 = maps.reshape(n, _H, _H)
    sums, maps_out = pl.pallas_call(
        _pool_body,
        in_specs=[pl.BlockSpec(memory_space=pl.ANY)],
        out_specs=[
            pl.BlockSpec(memory_space=pltpu.MemorySpace.VMEM),
            pl.BlockSpec(memory_space=pl.ANY),
        ],
        out_shape=[
            jax.ShapeDtypeStruct((n, 1), jnp.float32),
            jax.ShapeDtypeStruct((n, _H, _H), jnp.float32),
        ],
        scratch_shapes=(
            [pltpu.VMEM((_R, _H, _H), jnp.float32)] * _NBUF
            + [pltpu.SemaphoreType.DMA] * (2 * _NBUF)
        ),
    )(maps3)
    x_sun = sums.reshape(_B, _A)

    mesh = plsc.VectorSubcoreMesh(core_axis_name="c", subcore_axis_name="s")
    gate = functools.partial(
        pl.kernel,
        mesh=mesh,
        out_type=jax.ShapeDtypeStruct((_B, 16), jnp.float32),
        scratch_types=[
            pltpu.VMEM((_B * _A + _AP,), jnp.float32),
            pltpu.VMEM((_AP,), jnp.float32),
            pltpu.VMEM((16,), jnp.float32),
        ],
        compiler_params=pltpu.CompilerParams(needs_layout_passes=False),
    )(_sc_gate_body)
    son = gate(sums.reshape(n), W2.reshape(_A))
    x_son = son[:, :9].T.reshape(9, _B, 1)
    return (x_sun, x_son, maps_out.reshape(_B, _A, _H, _H))
